# baseline (speedup 1.0000x reference)
import functools

import jax
import jax.numpy as jnp
from jax import lax
from jax.experimental import pallas as pl
from jax.experimental.pallas import tpu as pltpu
from jax.experimental.pallas import tpu_sc as plsc

VOCAB = 1000000
EMBED_DIM = 32
BATCH = 16384
HIST = 50
NC, NS = 2, 16
NW = NC * NS
BW = BATCH // NW
BT = BW // 128
NBUF = 2

VT_TOTAL = VOCAB // 128
VT_PER_W = -(-VT_TOTAL // NW)
V_TAIL = VOCAB - VT_TOTAL * 128
PACKED_ROWS = VOCAB // 4


@functools.partial(
    pl.kernel,
    out_type=jax.ShapeDtypeStruct((PACKED_ROWS, 128), jnp.float32),
    mesh=plsc.VectorSubcoreMesh(core_axis_name="c", subcore_axis_name="s"),
    compiler_params=pltpu.CompilerParams(use_tc_tiling_on_sc=True,
                                         needs_layout_passes=False),
    scratch_types=(
        [pltpu.VMEM((NBUF, EMBED_DIM, 128), jnp.float32),
         pltpu.VMEM((NBUF, EMBED_DIM, 128), jnp.float32),
         pltpu.VMEM((V_TAIL, 128), jnp.float32)]
        + [pltpu.SemaphoreType.DMA] * (2 * NBUF)
    ),
)
def _sc_repack(tt_hbm, tail_hbm, packed_hbm, in_v, out_v, tail_v, *sems):
    isems = sems[:NBUF]
    osems = sems[NBUF:]
    wid = lax.axis_index("s") * NC + lax.axis_index("c")
    start = wid * VT_PER_W
    end = jnp.minimum(start + VT_PER_W, VT_TOTAL)
    n_my = end - start

    def in_cp(vt, b):
        return pltpu.make_async_copy(
            tt_hbm.at[:, pl.ds(vt * 128, 128)], in_v.at[b], isems[b])

    def out_cp(vt, b):
        return pltpu.make_async_copy(
            out_v.at[b], packed_hbm.at[pl.ds(vt * EMBED_DIM, EMBED_DIM), :],
            osems[b])

    lanes = lax.iota(jnp.int32, 16)

    row_m = [4 * m + lanes // 4 for m in range(8)]
    col_base = (lanes % 4) * EMBED_DIM

    def transpose_block(b):
        @plsc.parallel_loop(0, EMBED_DIM, step=1, unroll=4)
        def e_body(e):
            cols = col_base + e
            for m in range(8):
                vals = in_v[b, e, pl.ds(16 * m, 16)]
                plsc.store_scatter(out_v.at[b], [row_m[m], cols], vals)

    @pl.when(n_my > 0)
    def _():
        in_cp(start, 0).start()

    def pair_body(p, carry):
        for b in range(NBUF):
            i = p * NBUF + b
            vt = start + i

            @pl.when(vt + 1 < end)
            def _():
                in_cp(vt + 1, 1 - b).start()

            @pl.when(vt < end)
            def _():
                in_cp(vt, b).wait()

                @pl.when(i >= NBUF)
                def _():
                    out_cp(vt - NBUF, b).wait()

                transpose_block(b)
                out_cp(vt, b).start()
        return carry

    lax.fori_loop(0, -(-VT_PER_W // NBUF), pair_body, 0)
    for b in range(NBUF):
        @pl.when(n_my > b)
        def _():
            out_cp(start, b).wait()

    @pl.when(wid == NW - 1)
    def _():
        pltpu.sync_copy(tail_hbm, tail_v)

        @plsc.parallel_loop(0, V_TAIL // 4, step=1, unroll=4)
        def tail_body(r):
            for k in range(8):
                rows_idx = jnp.full((16,), 4 * r + k // 2, jnp.int32)
                cols = lanes + 16 * (k % 2)
                vals = plsc.load_gather(tail_v, [rows_idx, cols])
                out_v[0, r, pl.ds(16 * k, 16)] = vals

        pltpu.sync_copy(
            out_v.at[0, pl.ds(0, V_TAIL // 4)],
            packed_hbm.at[pl.ds(VT_TOTAL * EMBED_DIM, V_TAIL // 4), :])


@functools.partial(
    pl.kernel,
    out_type=jax.ShapeDtypeStruct((HIST, EMBED_DIM // 8, BATCH // 128, 8, 128),
                                  jnp.float32),
    mesh=plsc.VectorSubcoreMesh(core_axis_name="c", subcore_axis_name="s"),
    compiler_params=pltpu.CompilerParams(use_tc_tiling_on_sc=False,
                                         needs_layout_passes=False),
    scratch_types=(
        [pltpu.VMEM((HIST, BW), jnp.int32),
         pltpu.VMEM((NBUF, BW, EMBED_DIM), jnp.float32),
         pltpu.VMEM((NBUF, EMBED_DIM // 8, BT, 8, 128), jnp.float32)]
        + [pltpu.SemaphoreType.DMA] * (2 * NBUF)
    ),
)
def _sc_gather(xt_hbm, table_hbm, out_hbm, idx_v, rows_v, tiles_v, *sems):
    gsems = sems[:NBUF]
    osems = sems[NBUF:]
    wid = lax.axis_index("s") * NC + lax.axis_index("c")
    b0 = wid * BW

    pltpu.sync_copy(xt_hbm.at[:, pl.ds(b0, BW)], idx_v)

    def gather_cp(t, b):
        return pltpu.make_async_copy(
            table_hbm.at[idx_v.at[t]], rows_v.at[b], gsems[b])

    def write_cp(t, b):
        return pltpu.make_async_copy(
            tiles_v.at[b], out_hbm.at[t, :, pl.ds(wid * BT, BT)], osems[b])

    lanes = lax.iota(jnp.int32, 16)

    c8_h = [2 * h + lanes // 8 for h in range(2)]
    cs_v = lanes % 8

    def transpose_rows(b):
        @plsc.parallel_loop(0, BW, step=1, unroll=8)
        def row_body(r):
            btv = jnp.full((16,), r // 128, jnp.int32)
            pos = jnp.full((16,), r % 128, jnp.int32)
            for h in range(2):
                vals = rows_v[b, r, pl.ds(16 * h, 16)]
                plsc.store_scatter(tiles_v.at[b], [c8_h[h], btv, cs_v, pos],
                                   vals)

    gather_cp(0, 0).start()

    def pair_body(r, carry):
        for b in range(NBUF):
            t = r * NBUF + b

            @pl.when(t + 1 < HIST)
            def _():
                gather_cp(t + 1, 1 - b).start()

            gather_cp(t, b).wait()

            @pl.when(t >= NBUF)
            def _():
                write_cp(t - NBUF, b).wait()

            transpose_rows(b)
            write_cp(t, b).start()
        return carry

    lax.fori_loop(0, HIST // NBUF, pair_body, 0)
    write_cp(HIST - 2, 0).wait()
    write_cp(HIST - 1, 1).wait()


def kernel(x, table):
    table_t = table.T
    tail = lax.slice(table, (VT_TOTAL * 128, 0), (VOCAB, EMBED_DIM))
    tail_p = jnp.pad(tail, ((0, 0), (0, 128 - EMBED_DIM)))
    packed = _sc_repack(table_t, tail_p)
    raw = _sc_gather(x.T, packed.reshape(VOCAB, EMBED_DIM))
    out = lax.transpose(raw, (2, 4, 0, 1, 3)).reshape(BATCH, HIST, EMBED_DIM)
    return out


# speedup vs baseline: 1.0817x; 1.0817x over previous
import functools

import jax
import jax.numpy as jnp
from jax import lax
from jax.experimental import pallas as pl
from jax.experimental.pallas import tpu as pltpu
from jax.experimental.pallas import tpu_sc as plsc

VOCAB = 1000000
EMBED_DIM = 32
BATCH = 16384
HIST = 50
NC, NS = 2, 16
NW = NC * NS
BW = BATCH // NW
BT = BW // 128
NBUF = 2

VT_TOTAL = VOCAB // 128
VT_PER_W = -(-VT_TOTAL // NW)
V_TAIL = VOCAB - VT_TOTAL * 128
PACKED_ROWS = VOCAB // 4


@functools.partial(
    pl.kernel,
    out_type=jax.ShapeDtypeStruct((PACKED_ROWS, 128), jnp.float32),
    mesh=plsc.VectorSubcoreMesh(core_axis_name="c", subcore_axis_name="s"),
    compiler_params=pltpu.CompilerParams(use_tc_tiling_on_sc=True,
                                         needs_layout_passes=False),
    scratch_types=(
        [pltpu.VMEM((NBUF, EMBED_DIM, 128), jnp.float32),
         pltpu.VMEM((NBUF, EMBED_DIM, 128), jnp.float32),
         pltpu.VMEM((V_TAIL, 128), jnp.float32)]
        + [pltpu.SemaphoreType.DMA] * (2 * NBUF)
    ),
)
def _sc_repack(tt_hbm, tail_hbm, packed_hbm, in_v, out_v, tail_v, *sems):
    isems = sems[:NBUF]
    osems = sems[NBUF:]
    wid = lax.axis_index("s") * NC + lax.axis_index("c")
    start = wid * VT_PER_W
    end = jnp.minimum(start + VT_PER_W, VT_TOTAL)
    n_my = end - start

    def in_cp(vt, b):
        return pltpu.make_async_copy(
            tt_hbm.at[:, pl.ds(vt * 128, 128)], in_v.at[b], isems[b])

    def out_cp(vt, b):
        return pltpu.make_async_copy(
            out_v.at[b], packed_hbm.at[pl.ds(vt * EMBED_DIM, EMBED_DIM), :],
            osems[b])

    lanes = lax.iota(jnp.int32, 16)

    rows_h = [lanes + 16 * h for h in range(2)]

    def transpose_block(b):
        @plsc.parallel_loop(0, EMBED_DIM, step=1, unroll=4)
        def row_body(r):
            for j in range(4):
                cols = jnp.full((16,), 4 * r + j, jnp.int32)
                for h in range(2):
                    vals = plsc.load_gather(in_v.at[b], [rows_h[h], cols])
                    out_v[b, r, pl.ds(32 * j + 16 * h, 16)] = vals

    @pl.when(n_my > 0)
    def _():
        in_cp(start, 0).start()

    def pair_body(p, carry):
        for b in range(NBUF):
            i = p * NBUF + b
            vt = start + i

            @pl.when(vt + 1 < end)
            def _():
                in_cp(vt + 1, 1 - b).start()

            @pl.when(vt < end)
            def _():
                in_cp(vt, b).wait()

                @pl.when(i >= NBUF)
                def _():
                    out_cp(vt - NBUF, b).wait()

                transpose_block(b)
                out_cp(vt, b).start()
        return carry

    lax.fori_loop(0, -(-VT_PER_W // NBUF), pair_body, 0)
    for b in range(NBUF):
        @pl.when(n_my > b)
        def _():
            out_cp(start, b).wait()

    @pl.when(wid == NW - 1)
    def _():
        pltpu.sync_copy(tail_hbm, tail_v)

        @plsc.parallel_loop(0, V_TAIL // 4, step=1, unroll=4)
        def tail_body(r):
            for k in range(8):
                rows_idx = jnp.full((16,), 4 * r + k // 2, jnp.int32)
                cols = lanes + 16 * (k % 2)
                vals = plsc.load_gather(tail_v, [rows_idx, cols])
                out_v[0, r, pl.ds(16 * k, 16)] = vals

        pltpu.sync_copy(
            out_v.at[0, pl.ds(0, V_TAIL // 4)],
            packed_hbm.at[pl.ds(VT_TOTAL * EMBED_DIM, V_TAIL // 4), :])


@functools.partial(
    pl.kernel,
    out_type=jax.ShapeDtypeStruct((HIST, EMBED_DIM // 8, BATCH // 128, 8, 128),
                                  jnp.float32),
    mesh=plsc.VectorSubcoreMesh(core_axis_name="c", subcore_axis_name="s"),
    compiler_params=pltpu.CompilerParams(use_tc_tiling_on_sc=False,
                                         needs_layout_passes=False),
    scratch_types=(
        [pltpu.VMEM((HIST, BW), jnp.int32),
         pltpu.VMEM((NBUF, BW, EMBED_DIM), jnp.float32),
         pltpu.VMEM((NBUF, EMBED_DIM // 8, BT, 8, 128), jnp.float32)]
        + [pltpu.SemaphoreType.DMA] * (2 * NBUF)
    ),
)
def _sc_gather(xt_hbm, table_hbm, out_hbm, idx_v, rows_v, tiles_v, *sems):
    gsems = sems[:NBUF]
    osems = sems[NBUF:]
    wid = lax.axis_index("s") * NC + lax.axis_index("c")
    b0 = wid * BW

    pltpu.sync_copy(xt_hbm.at[:, pl.ds(b0, BW)], idx_v)

    def gather_cp(t, b):
        return pltpu.make_async_copy(
            table_hbm.at[idx_v.at[t]], rows_v.at[b], gsems[b])

    def write_cp(t, b):
        return pltpu.make_async_copy(
            tiles_v.at[b], out_hbm.at[t, :, pl.ds(wid * BT, BT)], osems[b])

    lanes = lax.iota(jnp.int32, 16)

    rows_bg = [lanes + 16 * i for i in range(BW // 16)]

    def transpose_rows(b):
        @plsc.parallel_loop(0, EMBED_DIM, step=1, unroll=4)
        def col_body(k):
            c8 = k // 8
            cs = k % 8
            cols = jnp.full((16,), k, jnp.int32)
            for bt in range(BT):
                for g in range(8):
                    vals = plsc.load_gather(rows_v.at[b],
                                            [rows_bg[bt * 8 + g], cols])
                    tiles_v[b, c8, bt, cs, pl.ds(g * 16, 16)] = vals

    gather_cp(0, 0).start()

    def pair_body(r, carry):
        for b in range(NBUF):
            t = r * NBUF + b

            @pl.when(t + 1 < HIST)
            def _():
                gather_cp(t + 1, 1 - b).start()

            gather_cp(t, b).wait()

            @pl.when(t >= NBUF)
            def _():
                write_cp(t - NBUF, b).wait()

            transpose_rows(b)
            write_cp(t, b).start()
        return carry

    lax.fori_loop(0, HIST // NBUF, pair_body, 0)
    write_cp(HIST - 2, 0).wait()
    write_cp(HIST - 1, 1).wait()


def kernel(x, table):
    table_t = table.T
    tail = lax.slice(table, (VT_TOTAL * 128, 0), (VOCAB, EMBED_DIM))
    tail_p = jnp.pad(tail, ((0, 0), (0, 128 - EMBED_DIM)))
    packed = _sc_repack(table_t, tail_p)
    raw = _sc_gather(x.T, packed.reshape(VOCAB, EMBED_DIM))
    out = lax.transpose(raw, (2, 4, 0, 1, 3)).reshape(BATCH, HIST, EMBED_DIM)
    return out
